# Initial kernel scaffold; baseline (speedup 1.0000x reference)
import jax
import jax.numpy as jnp
from jax.experimental import pallas as pl


def kernel(x, edge_index, counts, use_counts, batch, atom_table, eps, W1, b1, W2, b2, gamma, beta, Wd0, bd0, Wd1, bd1, Wd2, bd2):
    raise NotImplementedError("write your pallas kernel here")


# speedup vs baseline: 3.1732x; 3.1732x over previous
import functools

import jax
import jax.numpy as jnp
from jax import lax
from jax.experimental import pallas as pl
from jax.experimental.pallas import tpu as pltpu
from jax.experimental.pallas import tpu_sc as plsc

N = 10000
NP = 10240
NT = 512
NBLK = NP // NT
D = 128
E = 320000
NC, NS = 2, 16
NW = NC * NS
ECH = 80
EPAD = NW * ECH * 128
G = 512
GP = 528
NCHUNK = NP // 128
RPT = NP // NS

_MESH = dict(core_axis_name="c", subcore_axis_name="s",
             num_cores=NC, num_subcores=NS)


def _sc_agg(h, src_r, dst_r):
    @functools.partial(
        pl.kernel,
        out_type=jax.ShapeDtypeStruct((NC, NP, D), jnp.float32),
        mesh=plsc.VectorSubcoreMesh(**_MESH),
        scratch_types=[
            pltpu.VMEM((ECH, 128), jnp.int32),
            pltpu.VMEM((ECH, 128), jnp.int32),
            pltpu.VMEM((128, D), jnp.float32),
            pltpu.VMEM((64, D), jnp.float32),
            pltpu.VMEM_SHARED((NP, D), jnp.float32),
            pltpu.SemaphoreType.DMA,
        ],
    )
    def k(h_hbm, src_hbm, dst_hbm, out, src_v, dst_v, rows_v, zbuf,
          agg_sp, sem):
        cid = lax.axis_index("c")
        sid = lax.axis_index("s")
        wid = cid * NS + sid

        def zb(i, c):
            for kk in range(D // 16):
                zbuf[i, pl.ds(kk * 16, 16)] = jnp.zeros((16,), jnp.float32)
            return c
        lax.fori_loop(0, 64, zb, 0)

        def zc(i, c):
            pltpu.sync_copy(zbuf, agg_sp.at[pl.ds(sid * RPT + i * 64, 64)])
            return c
        lax.fori_loop(0, RPT // 64, zc, 0)
        plsc.subcore_barrier()

        pltpu.sync_copy(src_hbm.at[wid], src_v)
        pltpu.sync_copy(dst_hbm.at[wid], dst_v)

        def body(j, c):
            pltpu.async_copy(h_hbm.at[src_v.at[j]], rows_v, sem).wait()
            pltpu.sync_copy(rows_v, agg_sp.at[dst_v.at[j]], add=True)
            return c
        lax.fori_loop(0, ECH, body, 0)
        plsc.subcore_barrier()

        sl = pl.ds(sid * RPT, RPT)
        pltpu.sync_copy(agg_sp.at[sl], out.at[cid, sl])

    return k(h, src_r, dst_r)


def _sc_readout(h, batch2):
    @functools.partial(
        pl.kernel,
        out_type=(
            jax.ShapeDtypeStruct((NC, G, D), jnp.float32),
            jax.ShapeDtypeStruct((NC, G, D), jnp.float32),
        ),
        mesh=plsc.VectorSubcoreMesh(**_MESH),
        scratch_types=[
            pltpu.VMEM((128, D), jnp.float32),
            pltpu.VMEM((1, 128), jnp.int32),
            pltpu.VMEM((128, D), jnp.float32),
            pltpu.VMEM((33, D), jnp.float32),
            pltpu.VMEM_SHARED((GP, D), jnp.float32),
            pltpu.VMEM_SHARED((GP, D), jnp.float32),
        ],
    )
    def k(h_hbm, b_hbm, sums_out, cnt_out, hbuf, bidx, obuf, zb,
          sums_sp, cnt_sp):
        cid = lax.axis_index("c")
        sid = lax.axis_index("s")
        wid = cid * NS + sid

        def ib(i, c):
            for kk in range(D // 16):
                obuf[i, pl.ds(kk * 16, 16)] = jnp.ones((16,), jnp.float32)
            return c
        lax.fori_loop(0, 128, ib, 0)

        def zbb(i, c):
            for kk in range(D // 16):
                zb[i, pl.ds(kk * 16, 16)] = jnp.zeros((16,), jnp.float32)
            return c
        lax.fori_loop(0, 33, zbb, 0)

        pltpu.sync_copy(zb, sums_sp.at[pl.ds(sid * 33, 33)])
        pltpu.sync_copy(zb, cnt_sp.at[pl.ds(sid * 33, 33)])
        plsc.subcore_barrier()

        for j in range(3):
            c = wid + NW * j

            @pl.when(c < NCHUNK)
            def _():
                pltpu.sync_copy(h_hbm.at[pl.ds(c * 128, 128)], hbuf)
                pltpu.sync_copy(b_hbm.at[c], bidx.at[0])
                pltpu.sync_copy(hbuf, sums_sp.at[bidx.at[0]], add=True)
                pltpu.sync_copy(obuf, cnt_sp.at[bidx.at[0]], add=True)

        plsc.subcore_barrier()
        rows = pl.ds(sid * 32, 32)
        pltpu.sync_copy(sums_sp.at[rows], sums_out.at[cid, rows])
        pltpu.sync_copy(cnt_sp.at[rows], cnt_out.at[cid, rows])

    return k(h, batch2)


def _tc_enc(x2, counts_p, ucf, table_pad):
    def body(x_ref, c_ref, uc_ref, tb_ref, o_ref):
        xv = x_ref[...]
        oh = (xv == lax.broadcasted_iota(jnp.int32, (NT, 128), 1))
        h0 = jnp.dot(oh.astype(jnp.float32), tb_ref[...],
                     preferred_element_type=jnp.float32,
                     precision=lax.Precision.HIGHEST)
        o_ref[...] = h0 + uc_ref[0, 0] * c_ref[...]

    return pl.pallas_call(
        body,
        grid=(NBLK,),
        in_specs=[
            pl.BlockSpec((NT, 1), lambda i: (i, 0)),
            pl.BlockSpec((NT, 1), lambda i: (i, 0)),
            pl.BlockSpec(memory_space=pltpu.SMEM),
            pl.BlockSpec((128, D), lambda i: (0, 0)),
        ],
        out_specs=pl.BlockSpec((NT, D), lambda i: (i, 0)),
        out_shape=jax.ShapeDtypeStruct((NP, D), jnp.float32),
    )(x2, counts_p, ucf, table_pad)


def _tc_mlp(h, aggs, eps_i, w1, b1_i, w2, b2_i):
    def body(h_ref, a0_ref, a1_ref, eps_ref, w1_ref, b1_ref, w2_ref, b2_ref,
             t2_ref, sums_ref):
        i = pl.program_id(0)
        hh = (1.0 + eps_ref[0, 0]) * h_ref[...] + (a0_ref[0] + a1_ref[0])
        t = jnp.dot(hh, w1_ref[...], preferred_element_type=jnp.float32,
                    precision=lax.Precision.HIGHEST) + b1_ref[...]
        t = jnp.maximum(t, 0.0)
        t2 = jnp.dot(t, w2_ref[...], preferred_element_type=jnp.float32,
                     precision=lax.Precision.HIGHEST) + b2_ref[...]
        rows = i * NT + lax.broadcasted_iota(jnp.int32, (NT, 1), 0)
        t2 = jnp.where(rows < N, t2, 0.0)
        t2_ref[...] = t2

        @pl.when(i == 0)
        def _():
            sums_ref[...] = jnp.zeros_like(sums_ref)

        sums_ref[0:1, :] += jnp.sum(t2, axis=0, keepdims=True)
        sums_ref[1:2, :] += jnp.sum(t2 * t2, axis=0, keepdims=True)

    return pl.pallas_call(
        body,
        grid=(NBLK,),
        in_specs=[
            pl.BlockSpec((NT, D), lambda i: (i, 0)),
            pl.BlockSpec((1, NT, D), lambda i: (0, i, 0)),
            pl.BlockSpec((1, NT, D), lambda i: (1, i, 0)),
            pl.BlockSpec(memory_space=pltpu.SMEM),
            pl.BlockSpec((D, D), lambda i: (0, 0)),
            pl.BlockSpec((1, D), lambda i: (0, 0)),
            pl.BlockSpec((D, D), lambda i: (0, 0)),
            pl.BlockSpec((1, D), lambda i: (0, 0)),
        ],
        out_specs=[
            pl.BlockSpec((NT, D), lambda i: (i, 0)),
            pl.BlockSpec((2, D), lambda i: (0, 0)),
        ],
        out_shape=[
            jax.ShapeDtypeStruct((NP, D), jnp.float32),
            jax.ShapeDtypeStruct((2, D), jnp.float32),
        ],
    )(h, aggs, aggs, eps_i, w1, b1_i, w2, b2_i)


def _tc_bn(t2, h, sums, g_i, be_i):
    def body(t2_ref, h_ref, s_ref, g_ref, b_ref, o_ref):
        i = pl.program_id(0)
        s = s_ref[...]
        mu = s[0:1, :] * (1.0 / N)
        var = s[1:2, :] * (1.0 / N) - mu * mu
        inv = lax.rsqrt(var + 1e-5)
        y = (t2_ref[...] - mu) * (inv * g_ref[...]) + b_ref[...]
        hn = h_ref[...] + jnp.maximum(y, 0.0)
        rows = i * NT + lax.broadcasted_iota(jnp.int32, (NT, 1), 0)
        o_ref[...] = jnp.where(rows < N, hn, 0.0)

    return pl.pallas_call(
        body,
        grid=(NBLK,),
        in_specs=[
            pl.BlockSpec((NT, D), lambda i: (i, 0)),
            pl.BlockSpec((NT, D), lambda i: (i, 0)),
            pl.BlockSpec((2, D), lambda i: (0, 0)),
            pl.BlockSpec((1, D), lambda i: (0, 0)),
            pl.BlockSpec((1, D), lambda i: (0, 0)),
        ],
        out_specs=pl.BlockSpec((NT, D), lambda i: (i, 0)),
        out_shape=jax.ShapeDtypeStruct((NP, D), jnp.float32),
    )(t2, h, sums, g_i, be_i)


def _tc_final(s0, s1, c0, c1, w0, b0, w1, b1, w2, b2):
    def body(s0r, s1r, c0r, c1r, w0r, b0r, w1r, b1r, w2r, b2r, o_ref):
        s = s0r[...] + s1r[...]
        cnt = c0r[...] + c1r[...]
        g = s / jnp.maximum(cnt[:, 0:1], 1.0)
        g = jnp.maximum(jnp.dot(g, w0r[...], preferred_element_type=jnp.float32,
                                precision=lax.Precision.HIGHEST) + b0r[...], 0.0)
        g = jnp.maximum(jnp.dot(g, w1r[...], preferred_element_type=jnp.float32,
                                precision=lax.Precision.HIGHEST) + b1r[...], 0.0)
        o_ref[...] = jnp.dot(g, w2r[...], preferred_element_type=jnp.float32,
                             precision=lax.Precision.HIGHEST) + b2r[...]

    return pl.pallas_call(
        body,
        out_shape=jax.ShapeDtypeStruct((G, 128), jnp.float32),
    )(s0, s1, c0, c1, w0, b0, w1, b1, w2, b2)


def kernel(x, edge_index, counts, use_counts, batch, atom_table, eps, W1, b1,
           W2, b2, gamma, beta, Wd0, bd0, Wd1, bd1, Wd2, bd2):
    A = atom_table.shape[0]
    L = W1.shape[0]

    x2 = jnp.pad(x, (0, NP - N), constant_values=A).reshape(NP, 1)
    counts_p = jnp.pad(counts.astype(jnp.float32), ((0, NP - N), (0, 0)))
    ucf = jnp.asarray(use_counts, jnp.float32).reshape(1, 1)
    table_pad = jnp.pad(atom_table, ((0, 128 - A), (0, 0)))

    src_p = jnp.concatenate(
        [edge_index[0], jnp.full((EPAD - E,), N, jnp.int32)]).reshape(
            NW, ECH, 128)
    dst_p = jnp.concatenate(
        [edge_index[1], jnp.zeros((EPAD - E,), jnp.int32)]).reshape(
            NW, ECH, 128)
    batch2 = jnp.pad(batch, (0, NP - N), constant_values=G).reshape(
        NCHUNK, 128)

    h = _tc_enc(x2, counts_p, ucf, table_pad)
    for i in range(L):
        aggs = _sc_agg(h, src_p, dst_p)
        t2, sums = _tc_mlp(h, aggs, eps[i].reshape(1, 1), W1[i],
                           b1[i].reshape(1, D), W2[i], b2[i].reshape(1, D))
        h = _tc_bn(t2, h, sums, gamma[i].reshape(1, D),
                   beta[i].reshape(1, D))

    sums_ro, cnt_ro = _sc_readout(h, batch2)

    w0p = jnp.pad(Wd0, ((0, 0), (0, 128 - Wd0.shape[1])))
    b0p = jnp.pad(bd0, (0, 128 - bd0.shape[0])).reshape(1, 128)
    w1p = jnp.pad(Wd1, ((0, 128 - Wd1.shape[0]), (0, 128 - Wd1.shape[1])))
    b1p = jnp.pad(bd1, (0, 128 - bd1.shape[0])).reshape(1, 128)
    w2p = jnp.pad(Wd2, ((0, 128 - Wd2.shape[0]), (0, 128 - Wd2.shape[1])))
    b2p = jnp.pad(bd2, (0, 128 - bd2.shape[0])).reshape(1, 128)

    out = _tc_final(sums_ro[0], sums_ro[1], cnt_ro[0], cnt_ro[1],
                    w0p, b0p, w1p, b1p, w2p, b2p)
    return out[:, :1]
